# baseline (speedup 1.0000x reference)
import jax
import jax.numpy as jnp
from jax.experimental import pallas as pl

_BLOCK = 10000


def _rowdot_body(u_ref, v_ref, m_ref, o_ref):
    w = u_ref[...] * v_ref[...]
    o_ref[...] = jax.lax.dot_general(
        w, m_ref[...], (((1,), (0,)), ((), ())),
        preferred_element_type=jnp.float32)


def kernel(user_emb, items_emb):
    n, d = user_emb.shape
    packed = 128 // d
    rows = n // packed
    u2 = user_emb.reshape(rows, 128)
    v2 = items_emb.reshape(rows, 128)
    seg = jnp.repeat(jnp.eye(packed, dtype=jnp.float32), d, axis=0)
    out = pl.pallas_call(
        _rowdot_body,
        grid=(rows // _BLOCK,),
        in_specs=[
            pl.BlockSpec((_BLOCK, 128), lambda i: (i, 0)),
            pl.BlockSpec((_BLOCK, 128), lambda i: (i, 0)),
            pl.BlockSpec((128, packed), lambda i: (0, 0)),
        ],
        out_specs=pl.BlockSpec((_BLOCK, packed), lambda i: (i, 0)),
        out_shape=jax.ShapeDtypeStruct((rows, packed), jnp.float32),
    )(u2, v2, seg)
    return out.reshape(n)
